# baseline (speedup 1.0000x reference)
import functools

import jax
import jax.numpy as jnp
from jax import lax
from jax.experimental import pallas as pl
from jax.experimental.pallas import tpu as pltpu
from jax.experimental.pallas import tpu_sc as plsc

_NUM_SAMPLES = 16
_B, _T, _C, _H, _W = 4, 64, 3, 224, 224
_PLANES = _B * _NUM_SAMPLES * _C
_NWORK = 32
_PER_W = _PLANES // _NWORK


def _sc_body(x_hbm, o_hbm, buf):
    wid = lax.axis_index("s") * 2 + lax.axis_index("c")
    for k in range(_PER_W):
        p = wid * _PER_W + k
        b = p // (_NUM_SAMPLES * _C)
        r = p % (_NUM_SAMPLES * _C)
        s = r // _C
        c = r % _C
        t = (s * (_T - 1)) // (_NUM_SAMPLES - 1)
        pltpu.sync_copy(x_hbm.at[b, t, c], buf)
        pltpu.sync_copy(buf, o_hbm.at[b, s, c])


@jax.jit
def kernel(x):
    mesh = plsc.VectorSubcoreMesh(core_axis_name="c", subcore_axis_name="s")
    f = functools.partial(
        pl.kernel,
        out_type=jax.ShapeDtypeStruct((_B, _NUM_SAMPLES, _C, _H, _W), x.dtype),
        mesh=mesh,
        scratch_types=[pltpu.VMEM((_H, _W), jnp.float32)],
    )(_sc_body)
    return f(x)


# speedup vs baseline: 1.0530x; 1.0530x over previous
import functools

import jax
import jax.numpy as jnp
from jax import lax
from jax.experimental import pallas as pl
from jax.experimental.pallas import tpu as pltpu
from jax.experimental.pallas import tpu_sc as plsc

_NUM_SAMPLES = 16
_B, _T, _C, _H, _W = 4, 64, 3, 224, 224
_PLANES = _B * _NUM_SAMPLES * _C
_NWORK = 32
_PER_W = _PLANES // _NWORK


def _sc_body(x_hbm, o_hbm, bufs, in_sems, out_sems):
    wid = lax.axis_index("s") * 2 + lax.axis_index("c")

    def coords(k):
        p = wid * _PER_W + k
        b = p // (_NUM_SAMPLES * _C)
        r = p % (_NUM_SAMPLES * _C)
        s = r // _C
        c = r % _C
        t = (s * (_T - 1)) // (_NUM_SAMPLES - 1)
        return b, s, c, t

    in_c = [None] * _PER_W
    out_c = [None] * _PER_W

    def start_out(k):
        b, s, c, _ = coords(k)
        kb = k % 2
        in_c[k].wait()
        out_c[k] = pltpu.async_copy(bufs.at[kb], o_hbm.at[b, s, c],
                                    out_sems.at[kb])

    for k in range(_PER_W):
        kb = k % 2
        if k >= 2:
            out_c[k - 2].wait()
        b, s, c, t = coords(k)
        in_c[k] = pltpu.async_copy(x_hbm.at[b, t, c], bufs.at[kb],
                                   in_sems.at[kb])
        if k >= 1:
            start_out(k - 1)
    start_out(_PER_W - 1)
    out_c[_PER_W - 2].wait()
    out_c[_PER_W - 1].wait()


@jax.jit
def kernel(x):
    mesh = plsc.VectorSubcoreMesh(core_axis_name="c", subcore_axis_name="s")
    f = functools.partial(
        pl.kernel,
        out_type=jax.ShapeDtypeStruct((_B, _NUM_SAMPLES, _C, _H, _W), x.dtype),
        mesh=mesh,
        scratch_types=[
            pltpu.VMEM((2, _H, _W), jnp.float32),
            pltpu.SemaphoreType.DMA((2,)),
            pltpu.SemaphoreType.DMA((2,)),
        ],
    )(_sc_body)
    return f(x)
